# baseline (speedup 1.0000x reference)
import jax
import jax.numpy as jnp
from jax.experimental import pallas as pl

_TB = 5


def _select_add_kernel(x_ref, pos_ref, fm_ref, me_ref, o_ref):
    xv = x_ref[...]
    me = me_ref[...][None]
    fm = jnp.transpose(fm_ref[...], (0, 2, 1))
    o_ref[...] = jnp.where(fm > 0, me, xv) + pos_ref[...]


def kernel(x, pos_embed, mask, masked_embed):
    B, T, D = x.shape
    xt = jnp.transpose(x, (1, 0, 2))
    post = jnp.transpose(pos_embed, (1, 0, 2))
    m = mask.reshape(B, T - 1).astype(jnp.float32)
    fm = jnp.pad(m.T, ((1, 0), (0, 0))).reshape(T, 1, B)

    out_t = pl.pallas_call(
        _select_add_kernel,
        grid=(T // _TB,),
        in_specs=[
            pl.BlockSpec((_TB, B, D), lambda t: (t, 0, 0)),
            pl.BlockSpec((_TB, 1, D), lambda t: (t, 0, 0)),
            pl.BlockSpec((_TB, 1, B), lambda t: (t, 0, 0)),
            pl.BlockSpec((1, D), lambda t: (0, 0)),
        ],
        out_specs=pl.BlockSpec((_TB, B, D), lambda t: (t, 0, 0)),
        out_shape=jax.ShapeDtypeStruct((T, B, D), x.dtype),
    )(xt, post, fm, masked_embed)
    return jnp.transpose(out_t, (1, 0, 2))


# speedup vs baseline: 1.6597x; 1.6597x over previous
import jax
import jax.numpy as jnp
from jax.experimental import pallas as pl

_TB = 41


def _select_add_kernel(x_ref, pos_ref, fm_ref, me_ref, o_ref):
    xv = x_ref[...]
    me = me_ref[...][None]
    fm = jnp.transpose(fm_ref[...], (0, 2, 1))
    o_ref[...] = jnp.where(fm, me, xv) + pos_ref[...]


def kernel(x, pos_embed, mask, masked_embed):
    B, T, D = x.shape
    xt = jnp.transpose(x, (1, 0, 2))
    post = jnp.transpose(pos_embed, (1, 0, 2))
    mt = mask.reshape(B, T - 1).T
    fm = jnp.pad(mt, ((1, 0), (0, 0))).reshape(T, 1, B)

    out_t = pl.pallas_call(
        _select_add_kernel,
        grid=(T // _TB,),
        in_specs=[
            pl.BlockSpec((_TB, B, D), lambda t: (t, 0, 0)),
            pl.BlockSpec((_TB, 1, D), lambda t: (t, 0, 0)),
            pl.BlockSpec((_TB, 1, B), lambda t: (t, 0, 0)),
            pl.BlockSpec((1, D), lambda t: (0, 0)),
        ],
        out_specs=pl.BlockSpec((_TB, B, D), lambda t: (t, 0, 0)),
        out_shape=jax.ShapeDtypeStruct((T, B, D), x.dtype),
    )(xt, post, fm, masked_embed)
    return jnp.transpose(out_t, (1, 0, 2))
